# baseline (device time: 179832 ns/iter reference)
import jax
import jax.numpy as jnp
from jax import lax
from jax.experimental import pallas as pl
from jax.experimental.pallas import tpu as pltpu

N_DEV = 16
B_LOC = 2
SQ = 256
SKV = 256
H_LOC = 4
DH = 64
D_MODEL = 512
D_HEADS = H_LOC * DH


def _ring_allgather_weights(wq_shard, wo_shard):

    def body(wq_ref, wo_ref, wq_out, wo_out, q_send, q_recv, o_send, o_recv):
        me = lax.axis_index("i")
        left = lax.rem(me + N_DEV - 1, N_DEV)
        right = lax.rem(me + 1, N_DEV)

        barrier = pltpu.get_barrier_semaphore()
        for nbr in (left, right):
            pl.semaphore_signal(
                barrier, inc=1, device_id=(nbr,),
                device_id_type=pl.DeviceIdType.MESH,
            )
        pl.semaphore_wait(barrier, 2)

        wq_out[me] = wq_ref[...].astype(jnp.bfloat16)
        wo_out[me] = wo_ref[...].astype(jnp.bfloat16)

        def hop(h, carry):
            c = lax.rem(me + 2 * N_DEV - h, N_DEV)
            rq = pltpu.make_async_remote_copy(
                src_ref=wq_out.at[c], dst_ref=wq_out.at[c],
                send_sem=q_send, recv_sem=q_recv,
                device_id=(right,), device_id_type=pl.DeviceIdType.MESH,
            )
            ro = pltpu.make_async_remote_copy(
                src_ref=wo_out.at[c], dst_ref=wo_out.at[c],
                send_sem=o_send, recv_sem=o_recv,
                device_id=(right,), device_id_type=pl.DeviceIdType.MESH,
            )
            rq.start()
            ro.start()
            rq.wait()
            ro.wait()
            return carry

        lax.fori_loop(0, N_DEV - 1, hop, 0)

    return pl.pallas_call(
        body,
        out_shape=[
            jax.ShapeDtypeStruct((N_DEV, D_MODEL, D_HEADS), jnp.bfloat16),
            jax.ShapeDtypeStruct((N_DEV, D_HEADS, D_MODEL), jnp.bfloat16),
        ],
        in_specs=[
            pl.BlockSpec(memory_space=pltpu.VMEM),
            pl.BlockSpec(memory_space=pltpu.VMEM),
        ],
        out_specs=[
            pl.BlockSpec(memory_space=pltpu.VMEM),
            pl.BlockSpec(memory_space=pltpu.VMEM),
        ],
        scratch_shapes=[
            pltpu.SemaphoreType.DMA,
            pltpu.SemaphoreType.DMA,
            pltpu.SemaphoreType.DMA,
            pltpu.SemaphoreType.DMA,
        ],
        compiler_params=pltpu.CompilerParams(collective_id=0),
    )(wq_shard, wo_shard)


def _attention_compute(x, wq_c, wo_c, k_t, v_t):

    def body(x_ref, wq_ref, wo_ref, k_ref, v_ref, out_ref):
        b = pl.program_id(0)
        j = pl.program_id(1)

        xb = x_ref[b].astype(jnp.bfloat16)
        q = jnp.dot(xb, wq_ref[j], preferred_element_type=jnp.float32)
        q = q.astype(jnp.bfloat16)

        qblk = lax.broadcasted_iota(jnp.int32, (SQ, SKV), 0) // 64
        kblk = lax.broadcasted_iota(jnp.int32, (SQ, SKV), 1) // 64
        mask = qblk == kblk

        ctxs = []
        for h in range(H_LOC):
            qh = q[:, h * DH:(h + 1) * DH]
            kh = k_ref[b, j * H_LOC + h]
            vh = v_ref[b, j * H_LOC + h]
            sc = lax.dot_general(
                qh, kh, (((1,), (1,)), ((), ())),
                preferred_element_type=jnp.float32,
            ) * 0.125
            sc = jnp.where(mask, sc, -1e9)
            m = jnp.max(sc, axis=1, keepdims=True)
            w = jnp.exp(sc - m)
            w = w / jnp.sum(w, axis=1, keepdims=True)
            ctxs.append(jnp.dot(w.astype(jnp.bfloat16), vh,
                                preferred_element_type=jnp.float32))
        ctx = jnp.concatenate(ctxs, axis=1).astype(jnp.bfloat16)

        part = jnp.dot(ctx, wo_ref[j], preferred_element_type=jnp.float32)

        @pl.when(j == 0)
        def _():
            out_ref[b] = part

        @pl.when(j > 0)
        def _():
            out_ref[b] = out_ref[b] + part

    return pl.pallas_call(
        body,
        grid=(B_LOC, N_DEV),
        out_shape=jax.ShapeDtypeStruct((B_LOC, SQ, D_MODEL), jnp.float32),
        in_specs=[pl.BlockSpec(memory_space=pltpu.VMEM)] * 5,
        out_specs=pl.BlockSpec(memory_space=pltpu.VMEM),
        compiler_params=pltpu.CompilerParams(
            dimension_semantics=("arbitrary", "arbitrary"),
        ),
    )(x, wq_c, wo_c, k_t, v_t)


def kernel(x, Wq, K_ext, V_ext, Wo):
    me = lax.axis_index("i")

    wq_c, wo_c = _ring_allgather_weights(Wq, Wo)

    k_loc = lax.dynamic_slice_in_dim(K_ext, me * B_LOC, B_LOC, axis=0)
    v_loc = lax.dynamic_slice_in_dim(V_ext, me * B_LOC, B_LOC, axis=0)
    k_t = jnp.transpose(k_loc, (0, 2, 1, 3)).astype(jnp.bfloat16)
    v_t = jnp.transpose(v_loc, (0, 2, 1, 3)).astype(jnp.bfloat16)

    return _attention_compute(x, wq_c, wo_c, k_t, v_t)


# device time: 93919 ns/iter; 1.9148x vs baseline; 1.9148x over previous
import jax
import jax.numpy as jnp
from jax import lax
from jax.experimental import pallas as pl
from jax.experimental.pallas import tpu as pltpu

N_DEV = 16
B_LOC = 2
SQ = 256
SKV = 256
H_LOC = 4
DH = 64
D_MODEL = 512
D_HEADS = H_LOC * DH

N_RIGHT = N_DEV // 2
N_LEFT = N_DEV - 1 - N_RIGHT


def _fused(x, wq_shard, wo_shard, k_t, v_t):
    def body(x_ref, wq_ref, wo_ref, k_ref, v_ref, out_ref,
             wq_c, wo_c, sems):
        me = lax.axis_index("i")
        left = lax.rem(me + N_DEV - 1, N_DEV)
        right = lax.rem(me + 1, N_DEV)

        barrier = pltpu.get_barrier_semaphore()
        for nbr in (left, right):
            pl.semaphore_signal(
                barrier, inc=1, device_id=(nbr,),
                device_id_type=pl.DeviceIdType.MESH,
            )
        pl.semaphore_wait(barrier, 2)

        wq_c[me] = wq_ref[...].astype(jnp.bfloat16)
        wo_c[me] = wo_ref[...].astype(jnp.bfloat16)

        xb = [x_ref[b].astype(jnp.bfloat16) for b in range(B_LOC)]

        qblk = lax.broadcasted_iota(jnp.int32, (SQ, SKV), 0) // 64
        kblk = lax.broadcasted_iota(jnp.int32, (SQ, SKV), 1) // 64
        mask = qblk == kblk

        out_ref[...] = jnp.zeros((B_LOC, SQ, D_MODEL), jnp.float32)

        def compute_chunk(c):
            for b in range(B_LOC):
                q = jnp.dot(xb[b], wq_c[c],
                            preferred_element_type=jnp.float32)
                q = q.astype(jnp.bfloat16)
                ctxs = []
                for h in range(H_LOC):
                    qh = q[:, h * DH:(h + 1) * DH]
                    kh = k_ref[b, c * H_LOC + h]
                    vh = v_ref[b, c * H_LOC + h]
                    sc = lax.dot_general(
                        qh, kh, (((1,), (1,)), ((), ())),
                        preferred_element_type=jnp.float32,
                    ) * 0.125
                    sc = jnp.where(mask, sc, -1e9)
                    m = jnp.max(sc, axis=1, keepdims=True)
                    w = jnp.exp(sc - m)
                    w = w / jnp.sum(w, axis=1, keepdims=True)
                    ctxs.append(jnp.dot(w.astype(jnp.bfloat16), vh,
                                        preferred_element_type=jnp.float32))
                ctx = jnp.concatenate(ctxs, axis=1).astype(jnp.bfloat16)
                part = jnp.dot(ctx, wo_c[c],
                               preferred_element_type=jnp.float32)
                out_ref[b] = out_ref[b] + part

        def make(c, dir_idx, tensor_idx, target):
            buf = wq_c if tensor_idx == 0 else wo_c
            return pltpu.make_async_remote_copy(
                src_ref=buf.at[c], dst_ref=buf.at[c],
                send_sem=sems.at[dir_idx, tensor_idx, 0],
                recv_sem=sems.at[dir_idx, tensor_idx, 1],
                device_id=(target,), device_id_type=pl.DeviceIdType.MESH,
            )

        def hop(s, carry):
            cr = lax.rem(me + 2 * N_DEV - (s - 1), N_DEV)
            cl = lax.rem(me + s - 1, N_DEV)
            rq = make(cr, 0, 0, right)
            ro = make(cr, 0, 1, right)
            lq = make(cl, 1, 0, left)
            lo = make(cl, 1, 1, left)
            rq.start()
            ro.start()

            @pl.when(s <= N_LEFT)
            def _():
                lq.start()
                lo.start()

            compute_chunk(cr)

            @pl.when(s > 1)
            def _():
                compute_chunk(cl)

            rq.wait()
            ro.wait()

            @pl.when(s <= N_LEFT)
            def _():
                lq.wait()
                lo.wait()

            return carry

        lax.fori_loop(1, N_RIGHT + 1, hop, 0)

        compute_chunk(lax.rem(me + N_DEV - N_RIGHT, N_DEV))

    return pl.pallas_call(
        body,
        out_shape=jax.ShapeDtypeStruct((B_LOC, SQ, D_MODEL), jnp.float32),
        in_specs=[pl.BlockSpec(memory_space=pltpu.VMEM)] * 5,
        out_specs=pl.BlockSpec(memory_space=pltpu.VMEM),
        scratch_shapes=[
            pltpu.VMEM((N_DEV, D_MODEL, D_HEADS), jnp.bfloat16),
            pltpu.VMEM((N_DEV, D_HEADS, D_MODEL), jnp.bfloat16),
            pltpu.SemaphoreType.DMA((2, 2, 2)),
        ],
        compiler_params=pltpu.CompilerParams(collective_id=0),
    )(x, wq_shard, wo_shard, k_t, v_t)


def kernel(x, Wq, K_ext, V_ext, Wo):
    me = lax.axis_index("i")

    k_loc = lax.dynamic_slice_in_dim(K_ext, me * B_LOC, B_LOC, axis=0)
    v_loc = lax.dynamic_slice_in_dim(V_ext, me * B_LOC, B_LOC, axis=0)
    k_t = jnp.transpose(k_loc, (0, 2, 1, 3)).astype(jnp.bfloat16)
    v_t = jnp.transpose(v_loc, (0, 2, 1, 3)).astype(jnp.bfloat16)

    return _fused(x, Wq, Wo, k_t, v_t)
